# baseline (speedup 1.0000x reference)
import functools

import jax
import jax.numpy as jnp
from jax import lax
from jax.experimental import pallas as pl
from jax.experimental.pallas import tpu as pltpu
from jax.experimental.pallas import tpu_sc as plsc

B, H = 128, 1024
G = 8
HB = 4 * H // G

NC, NS = 2, 16
NW_USED = 16
ROWS = B // NW_USED

@functools.cache
def _make_sc_gather():
    mesh = plsc.VectorSubcoreMesh(core_axis_name="c", subcore_axis_name="s")

    @functools.partial(
        pl.kernel,
        mesh=mesh,
        out_type=jax.ShapeDtypeStruct((B, H), jnp.float32),
        scratch_types=[
            pltpu.VMEM((ROWS,), jnp.int32),
            pltpu.VMEM((ROWS, H), jnp.float32),
            pltpu.SemaphoreType.DMA,
        ],
    )
    def _sc_gather(table_hbm, idx_hbm, out_hbm, idx_v, rows_v, sem):
        wid = lax.axis_index("s") * NC + lax.axis_index("c")

        @pl.when(wid < NW_USED)
        def _():
            base = wid * ROWS
            pltpu.sync_copy(idx_hbm.at[pl.ds(base, ROWS)], idx_v)
            pltpu.async_copy(table_hbm.at[idx_v], rows_v, sem).wait()
            pltpu.sync_copy(rows_v, out_hbm.at[pl.ds(base, ROWS)])

    return _sc_gather


def _lstm_body(emb_ref, h0_ref, c0_ref, wih_ref, whh_ref, bih_ref, bhh_ref,
               h_out, c_out, acc_ref):
    k = pl.program_id(0)
    dn = (((1,), (1,)), ((), ()))
    pre = lax.dot_general(emb_ref[...], wih_ref[...], dn,
                          preferred_element_type=jnp.float32)
    pre += lax.dot_general(h0_ref[...], whh_ref[...], dn,
                           preferred_element_type=jnp.float32)
    pre += bih_ref[...] + bhh_ref[...]
    quarter = k // (G // 4)
    act = jnp.where(quarter == 2, jnp.tanh(pre), jax.nn.sigmoid(pre))
    acc_ref[:, pl.ds(k * HB, HB)] = act

    @pl.when(k == G - 1)
    def _():
        i = acc_ref[:, 0:H]
        f = acc_ref[:, H:2 * H]
        g = acc_ref[:, 2 * H:3 * H]
        o = acc_ref[:, 3 * H:4 * H]
        c = f * c0_ref[...] + i * g
        c_out[...] = c
        h_out[...] = o * jnp.tanh(c)


_lstm = pl.pallas_call(
    _lstm_body,
    grid=(G,),
    in_specs=[
        pl.BlockSpec((B, H), lambda k: (0, 0)),
        pl.BlockSpec((B, H), lambda k: (0, 0)),
        pl.BlockSpec((B, H), lambda k: (0, 0)),
        pl.BlockSpec((HB, H), lambda k: (k, 0)),
        pl.BlockSpec((HB, H), lambda k: (k, 0)),
        pl.BlockSpec((1, HB), lambda k: (0, k)),
        pl.BlockSpec((1, HB), lambda k: (0, k)),
    ],
    out_specs=[
        pl.BlockSpec((B, H), lambda k: (0, 0)),
        pl.BlockSpec((B, H), lambda k: (0, 0)),
    ],
    out_shape=[
        jax.ShapeDtypeStruct((B, H), jnp.float32),
        jax.ShapeDtypeStruct((B, H), jnp.float32),
    ],
    scratch_shapes=[pltpu.VMEM((B, 4 * H), jnp.float32)],
    compiler_params=pltpu.CompilerParams(dimension_semantics=("arbitrary",)),
)


def kernel(x, hidden, cell, table, W_ih, W_hh, b_ih, b_hh):
    emb = _make_sc_gather()(table, x)
    h, c = _lstm(emb, hidden[0], cell[0], W_ih, W_hh,
                 b_ih.reshape(1, 4 * H), b_hh.reshape(1, 4 * H))
    return (h[None], h[None], c[None])


# speedup vs baseline: 1.7405x; 1.7405x over previous
import functools

import jax
import jax.numpy as jnp
from jax import lax
from jax.experimental import pallas as pl
from jax.experimental.pallas import tpu as pltpu
from jax.experimental.pallas import tpu_sc as plsc

B, H = 128, 1024
G = 8
HB = 4 * H // G

NC, NS = 2, 16
NW_USED = 16
ROWS = B // NW_USED

@functools.cache
def _make_sc_gather():
    mesh = plsc.VectorSubcoreMesh(core_axis_name="c", subcore_axis_name="s")

    @functools.partial(
        pl.kernel,
        mesh=mesh,
        out_type=jax.ShapeDtypeStruct((B, H), jnp.float32),
        scratch_types=[
            pltpu.VMEM((ROWS,), jnp.int32),
            pltpu.VMEM((ROWS, H), jnp.float32),
            pltpu.SemaphoreType.DMA,
        ],
    )
    def _sc_gather(table_hbm, idx_hbm, out_hbm, idx_v, rows_v, sem):
        wid = lax.axis_index("s") * NC + lax.axis_index("c")

        @pl.when(wid < NW_USED)
        def _():
            base = wid * ROWS
            pltpu.sync_copy(idx_hbm.at[pl.ds(base, ROWS)], idx_v)
            pltpu.async_copy(table_hbm.at[idx_v], rows_v, sem).wait()
            pltpu.sync_copy(rows_v, out_hbm.at[pl.ds(base, ROWS)])

    return _sc_gather


def _lstm_body(emb_ref, h0_ref, c0_ref, wih_ref, whh_ref, bih_ref, bhh_ref,
               h_out, c_out, acc_ref):
    k = pl.program_id(0)
    dn = (((1,), (1,)), ((), ()))
    pre = lax.dot_general(emb_ref[...], wih_ref[...], dn,
                          preferred_element_type=jnp.float32)
    pre += lax.dot_general(h0_ref[...], whh_ref[...], dn,
                           preferred_element_type=jnp.float32)
    pre += bih_ref[...] + bhh_ref[...]
    quarter = k // (G // 4)
    act = jnp.where(quarter == 2, jnp.tanh(pre), jax.nn.sigmoid(pre))
    acc_ref[:, pl.ds(k * HB, HB)] = act

    @pl.when(k == G - 1)
    def _():
        i = acc_ref[:, 0:H]
        f = acc_ref[:, H:2 * H]
        g = acc_ref[:, 2 * H:3 * H]
        o = acc_ref[:, 3 * H:4 * H]
        c = f * c0_ref[...] + i * g
        c_out[...] = c
        h_out[...] = o * jnp.tanh(c)


_lstm = pl.pallas_call(
    _lstm_body,
    grid=(G,),
    in_specs=[
        pl.BlockSpec((B, H), lambda k: (0, 0)),
        pl.BlockSpec((B, H), lambda k: (0, 0)),
        pl.BlockSpec((B, H), lambda k: (0, 0)),
        pl.BlockSpec((HB, H), lambda k: (k, 0)),
        pl.BlockSpec((HB, H), lambda k: (k, 0)),
        pl.BlockSpec((1, HB), lambda k: (0, k)),
        pl.BlockSpec((1, HB), lambda k: (0, k)),
    ],
    out_specs=[
        pl.BlockSpec((B, H), lambda k: (0, 0)),
        pl.BlockSpec((B, H), lambda k: (0, 0)),
    ],
    out_shape=[
        jax.ShapeDtypeStruct((B, H), jnp.float32),
        jax.ShapeDtypeStruct((B, H), jnp.float32),
    ],
    scratch_shapes=[pltpu.VMEM((B, 4 * H), jnp.float32)],
    compiler_params=pltpu.CompilerParams(dimension_semantics=("arbitrary",)),
)


def kernel(x, hidden, cell, table, W_ih, W_hh, b_ih, b_hh):
    emb = jnp.take(table, x, axis=0)
    h, c = _lstm(emb, hidden[0], cell[0], W_ih, W_hh,
                 b_ih.reshape(1, 4 * H), b_hh.reshape(1, 4 * H))
    return (h[None], h[None], c[None])
